# baseline (speedup 1.0000x reference)
import jax
import jax.numpy as jnp
from jax.experimental import pallas as pl
from jax.experimental.pallas import tpu as pltpu

_N = 10000
_NP = 10240
_B = 512
_K = 8
_PAD_COORD = 1.0e3


def _eighth_smallest(d2, rows):
    prev = jnp.full((rows, 1), -jnp.inf, dtype=jnp.float32)
    for _ in range(_K):
        cand = jnp.where(d2 > prev, d2, jnp.inf)
        prev = jnp.min(cand, axis=1, keepdims=True)
    return prev


def _splash_kernel(coords_ref, means_t_ref, feats_ref, log_covs_t_ref,
                   out_ref, const_ref):
    q = coords_ref[...]
    mt = means_t_ref[...]

    @pl.when(pl.program_id(0) == 0)
    def _init_consts():
        m_sq0 = (mt[0:1, :] * mt[0:1, :] + mt[1:2, :] * mt[1:2, :]
                 + mt[2:3, :] * mt[2:3, :])
        lc = log_covs_t_ref[...]
        cmean = (jnp.exp(lc[0:1, :]) + jnp.exp(lc[1:2, :])
                 + jnp.exp(lc[2:3, :])) * (1.0 / 3.0)
        const_ref[0:1, :] = m_sq0
        const_ref[1:2, :] = (-0.5 * 1.4426950408889634) / (cmean + 1e-12)

    m_sq = const_ref[0:1, :]
    neg_half_inv_var = const_ref[1:2, :]

    q_sq = (q[:, 0:1] * q[:, 0:1] + q[:, 1:2] * q[:, 1:2]
            + q[:, 2:3] * q[:, 2:3])
    mdot = jax.lax.dot_general(
        q, mt, (((1,), (0,)), ((), ())),
        preferred_element_type=jnp.float32)
    d2 = (q_sq - 2.0 * mdot) + m_sq

    rows = q.shape[0]
    inf = jnp.float32(jnp.inf)
    a1 = jnp.full((rows, 128), inf, dtype=jnp.float32)
    a2 = a1
    a3 = a1
    a4 = a1
    for j in range(_NP // 128):
        v = d2[:, j * 128:(j + 1) * 128]
        t1 = jnp.minimum(a1, v)
        v = jnp.maximum(a1, v)
        a1 = t1
        t2 = jnp.minimum(a2, v)
        v = jnp.maximum(a2, v)
        a2 = t2
        t3 = jnp.minimum(a3, v)
        v = jnp.maximum(a3, v)
        a3 = t3
        a4 = jnp.minimum(a4, v)
    surv = jnp.concatenate([a1, a2, a3], axis=1)
    t_cand = _eighth_smallest(surv, rows)

    bad = jnp.any(a4 <= t_cand)

    p = jnp.exp2(jnp.maximum(d2, 0.0) * neg_half_inv_var)
    feats = feats_ref[...]
    nf = feats.shape[1] - 1

    w = jnp.where(d2 <= t_cand, p, 0.0)
    acc = jax.lax.dot_general(
        w, feats, (((1,), (0,)), ((), ())),
        preferred_element_type=jnp.float32)
    out_ref[...] = acc[:, :nf] / (acc[:, nf:] + 1e-8)

    @pl.when(bad)
    def _fallback():
        thresh = _eighth_smallest(d2, rows)
        w2 = jnp.where(d2 <= thresh, p, 0.0)
        acc2 = jax.lax.dot_general(
            w2, feats, (((1,), (0,)), ((), ())),
            preferred_element_type=jnp.float32)
        out_ref[...] = acc2[:, :nf] / (acc2[:, nf:] + 1e-8)


def kernel(coords, means, feats, log_covs):
    qn, f = coords.shape[0], feats.shape[1]
    n = means.shape[0]
    pad = _NP - n
    coords8 = jnp.pad(coords, ((0, 0), (0, 5)))
    means_t = jnp.pad(means.T, ((0, 0), (0, pad)),
                      constant_values=_PAD_COORD)
    means_t8 = jnp.pad(means_t, ((0, 5), (0, 0)))
    feats_p = jnp.concatenate(
        [jnp.pad(feats, ((0, pad), (0, 0))),
         jnp.ones((_NP, 1), jnp.float32)], axis=1)
    log_covs_t = jnp.pad(log_covs.T, ((0, 0), (0, pad)))

    grid = qn // _B
    return pl.pallas_call(
        _splash_kernel,
        grid=(grid,),
        in_specs=[
            pl.BlockSpec((_B, 8), lambda i: (i, 0)),
            pl.BlockSpec((8, _NP), lambda i: (0, 0)),
            pl.BlockSpec((_NP, f + 1), lambda i: (0, 0)),
            pl.BlockSpec((3, _NP), lambda i: (0, 0)),
        ],
        out_specs=pl.BlockSpec((_B, f), lambda i: (i, 0)),
        out_shape=jax.ShapeDtypeStruct((qn, f), jnp.float32),
        scratch_shapes=[pltpu.VMEM((2, _NP), jnp.float32)],
    )(coords8, means_t8, feats_p, log_covs_t)


# speedup vs baseline: 1.0266x; 1.0266x over previous
import jax
import jax.numpy as jnp
from jax.experimental import pallas as pl
from jax.experimental.pallas import tpu as pltpu

_N = 10000
_NP = 10240
_B = 256
_K = 8
_PAD_COORD = 1.0e3


def _eighth_smallest(d2, rows):
    prev = jnp.full((rows, 1), -jnp.inf, dtype=jnp.float32)
    for _ in range(_K):
        cand = jnp.where(d2 > prev, d2, jnp.inf)
        prev = jnp.min(cand, axis=1, keepdims=True)
    return prev


def _splash_kernel(coords_ref, means_t_ref, feats_ref, log_covs_t_ref,
                   out_ref, const_ref):
    q = coords_ref[...]
    mt = means_t_ref[...]

    @pl.when(pl.program_id(0) == 0)
    def _init_consts():
        m_sq0 = 0.25 * (mt[0:1, :] * mt[0:1, :] + mt[1:2, :] * mt[1:2, :]
                        + mt[2:3, :] * mt[2:3, :])
        lc = log_covs_t_ref[...]
        cmean = (jnp.exp(lc[0:1, :]) + jnp.exp(lc[1:2, :])
                 + jnp.exp(lc[2:3, :])) * (1.0 / 3.0)
        const_ref[0:1, :] = m_sq0
        const_ref[1:2, :] = (-0.5 * 1.4426950408889634) / (cmean + 1e-12)

    m_sq = const_ref[0:1, :]
    neg_half_inv_var = const_ref[1:2, :]

    q_sq = (q[:, 0:1] * q[:, 0:1] + q[:, 1:2] * q[:, 1:2]
            + q[:, 2:3] * q[:, 2:3])
    mdot = jax.lax.dot_general(
        q, mt, (((1,), (0,)), ((), ())),
        preferred_element_type=jnp.float32)
    d2 = (q_sq + mdot) + m_sq

    rows = q.shape[0]
    inf = jnp.float32(jnp.inf)
    a1 = jnp.full((rows, 128), inf, dtype=jnp.float32)
    a2 = a1
    a3 = a1
    a4 = a1
    for j in range(_NP // 128):
        v = d2[:, j * 128:(j + 1) * 128]
        t1 = jnp.minimum(a1, v)
        v = jnp.maximum(a1, v)
        a1 = t1
        t2 = jnp.minimum(a2, v)
        v = jnp.maximum(a2, v)
        a2 = t2
        t3 = jnp.minimum(a3, v)
        v = jnp.maximum(a3, v)
        a3 = t3
        a4 = jnp.minimum(a4, v)
    surv = jnp.concatenate([a1, a2, a3], axis=1)
    t_cand = _eighth_smallest(surv, rows)

    bad = jnp.any(a4 <= t_cand)

    p = jnp.exp2(jnp.maximum(d2, 0.0) * neg_half_inv_var)
    feats = feats_ref[...]
    nf = feats.shape[1] - 1

    w = jnp.where(d2 <= t_cand, p, 0.0)
    acc = jax.lax.dot_general(
        w, feats, (((1,), (0,)), ((), ())),
        preferred_element_type=jnp.float32)
    out_ref[...] = acc[:, :nf] / (acc[:, nf:] + 1e-8)

    @pl.when(bad)
    def _fallback():
        thresh = _eighth_smallest(d2, rows)
        w2 = jnp.where(d2 <= thresh, p, 0.0)
        acc2 = jax.lax.dot_general(
            w2, feats, (((1,), (0,)), ((), ())),
            preferred_element_type=jnp.float32)
        out_ref[...] = acc2[:, :nf] / (acc2[:, nf:] + 1e-8)


def kernel(coords, means, feats, log_covs):
    qn, f = coords.shape[0], feats.shape[1]
    n = means.shape[0]
    pad = _NP - n
    coords8 = jnp.pad(coords, ((0, 0), (0, 5)))
    means_t = jnp.pad(means.T * -2.0, ((0, 0), (0, pad)),
                      constant_values=-2.0 * _PAD_COORD)
    means_t8 = jnp.pad(means_t, ((0, 5), (0, 0)))
    feats_p = jnp.concatenate(
        [jnp.pad(feats, ((0, pad), (0, 0))),
         jnp.ones((_NP, 1), jnp.float32)], axis=1)
    log_covs_t = jnp.pad(log_covs.T, ((0, 0), (0, pad)))

    grid = qn // _B
    return pl.pallas_call(
        _splash_kernel,
        grid=(grid,),
        in_specs=[
            pl.BlockSpec((_B, 8), lambda i: (i, 0)),
            pl.BlockSpec((8, _NP), lambda i: (0, 0)),
            pl.BlockSpec((_NP, f + 1), lambda i: (0, 0)),
            pl.BlockSpec((3, _NP), lambda i: (0, 0)),
        ],
        out_specs=pl.BlockSpec((_B, f), lambda i: (i, 0)),
        out_shape=jax.ShapeDtypeStruct((qn, f), jnp.float32),
        scratch_shapes=[pltpu.VMEM((2, _NP), jnp.float32)],
    )(coords8, means_t8, feats_p, log_covs_t)
